# baseline (speedup 1.0000x reference)
import jax
import jax.numpy as jnp
from jax import lax
from jax.experimental import pallas as pl
from jax.experimental.pallas import tpu as pltpu
from jax.experimental.pallas import tpu_sc as plsc

B, H, W = 8, 512, 512
HW = H * W
N = B * HW
F = 100000
D = 16
NW = 32
NPW = N // NW
CH = 1024
RPC = CH // W
SUB = CH // 128
NCHUNK = NPW // CH
WPB = HW // NPW


def _sc_body(pix_hbm, bary_hbm, table_hbm, out_hbm,
             i0, i1, g0, g1, b0, b1,
             o00, o01, o02, o10, o11, o12,
             gs0, gs1, is0, is1, bs0, bs1, os0, os1):
    I = (i0, i1); G = (g0, g1); BV = (b0, b1)
    O = ((o00, o01, o02), (o10, o11, o12))
    GS = (gs0, gs1); IS = (is0, is1); BS = (bs0, bs1); OS = (os0, os1)

    cid = lax.axis_index("c")
    sid = lax.axis_index("s")
    wid = sid * 2 + cid
    b = wid // WPB
    inoff = (wid % WPB) * NPW
    h_base = inoff // W

    iota = lax.iota(jnp.int32, 16)

    def idx_start(t, s):
        h0 = pl.multiple_of(h_base + t * RPC, RPC)
        pltpu.async_copy(pix_hbm.at[b, pl.ds(h0, RPC)], I[s], IS[s])

    def idx_wait(s):
        pltpu.make_async_copy(
            pix_hbm.at[0, pl.ds(0, RPC)], I[s], IS[s]).wait()

    def bary_start(t, s):
        h0 = pl.multiple_of(h_base + t * RPC, RPC)
        pltpu.async_copy(bary_hbm.at[b, pl.ds(h0, RPC)], BV[s], BS[s])

    def bary_wait(s):
        pltpu.make_async_copy(
            bary_hbm.at[0, pl.ds(0, RPC)], BV[s], BS[s]).wait()

    def gathers_start(s):
        for j in range(SUB):
            idx_ref = I[s].at[j // 4, pl.ds((j % 4) * 128, 128)]
            pltpu.async_copy(table_hbm.at[idx_ref], G[s].at[j], GS[s])

    def gathers_wait(s):
        for j in range(SUB):
            idx_ref = I[s].at[j // 4, pl.ds((j % 4) * 128, 128)]
            pltpu.make_async_copy(
                table_hbm.at[idx_ref], G[s].at[j], GS[s]).wait()

    def out_start(t, s):
        dst0 = b * (3 * HW) + inoff + t * CH
        for c in range(3):
            pltpu.async_copy(
                O[s][c],
                out_hbm.at[pl.ds(pl.multiple_of(dst0 + c * HW, CH), CH)],
                OS[s])

    def out_wait(s):
        for c in range(3):
            pltpu.make_async_copy(
                O[s][c], out_hbm.at[pl.ds(0, CH)], OS[s]).wait()

    def compute(s):
        for j in range(SUB):
            for k in range(8):
                p0 = j * 128 + k * 16
                r, w0 = divmod(p0, W)
                f = I[s][r, pl.ds(w0, 16)]
                mask = f > 0
                gjk = G[s].at[j, pl.ds(k * 16, 16)]
                bw = [BV[s][r, v, 0, pl.ds(w0, 16)] for v in range(3)]
                for c in range(3):
                    gg = [plsc.load_gather(
                        gjk, [iota, jnp.full((16,), 3 * v + c, jnp.int32)])
                        for v in range(3)]
                    acc = bw[0] * gg[0] + bw[1] * gg[1] + bw[2] * gg[2]
                    O[s][c][pl.ds(p0, 16)] = jnp.where(
                        mask, acc, jnp.zeros_like(acc))

    idx_start(0, 0)
    idx_wait(0)
    gathers_start(0)
    bary_start(0, 0)
    idx_start(1, 1)

    def body(t2, _):
        for par in range(2):
            s = par
            t = t2 * 2 + par
            nxt = s ^ 1

            @pl.when(t + 1 < NCHUNK)
            def _():
                idx_wait(nxt)
                gathers_start(nxt)
                bary_start(t + 1, nxt)

            gathers_wait(s)
            bary_wait(s)

            @pl.when(t >= 2)
            def _():
                out_wait(s)

            compute(s)
            out_start(t, s)

            @pl.when(t + 2 < NCHUNK)
            def _():
                idx_start(t + 2, s)
        return ()

    lax.fori_loop(0, NCHUNK // 2, body, (), unroll=False)
    out_wait(0)
    out_wait(1)


@jax.jit
def _texture_shade(pix3, bary5, table16):
    mesh = plsc.VectorSubcoreMesh(core_axis_name="c", subcore_axis_name="s")
    k = pl.kernel(
        _sc_body,
        out_type=jax.ShapeDtypeStruct((B * 3 * HW,), jnp.float32),
        mesh=mesh,
        compiler_params=pltpu.CompilerParams(
            needs_layout_passes=False, use_tc_tiling_on_sc=False),
        scratch_types=(
            [pltpu.VMEM((RPC, W), jnp.int32)] * 2
            + [pltpu.VMEM((SUB, 128, D), jnp.float32)] * 2
            + [pltpu.VMEM((RPC, 3, 1, W), jnp.float32)] * 2
            + [pltpu.VMEM((CH,), jnp.float32)] * 6
            + [pltpu.SemaphoreType.DMA] * 8
        ),
    )
    return k(pix3, bary5, table16)


def kernel(pix_to_face, bary_coords, face_verts_colors):
    pix3 = pix_to_face.astype(jnp.int32).reshape(B, H, W)
    bary5 = bary_coords.transpose(0, 1, 4, 3, 2)
    table16 = jnp.pad(
        face_verts_colors.reshape(F, 9), ((0, 0), (0, D - 9)))
    out = _texture_shade(pix3, bary5, table16)
    return out.reshape(B, 3, H, W)
